# baseline (speedup 1.0000x reference)
import functools

import jax
import jax.numpy as jnp
from jax import lax
from jax.experimental import pallas as pl
from jax.experimental.pallas import tpu as pltpu
from jax.experimental.pallas import tpu_sc as plsc

ROWS_PER_GATHER = 128
PAD_DIM = 128
NBUF = 8
PREFETCH = 4


@functools.lru_cache(maxsize=None)
def _make_sc_gather(n_b: int, n_tok: int, embed_num: int, embed_dim: int):
    info = plsc.get_sparse_core_info()
    nc, ns = info.num_cores, info.num_subcores
    nw = nc * ns
    n_rows = n_b * n_tok
    rows_per_w = n_rows // nw
    chunks = rows_per_w // ROWS_PER_GATHER
    rpg = ROWS_PER_GATHER
    cpb = n_tok // rpg

    mesh = plsc.VectorSubcoreMesh(core_axis_name="c", subcore_axis_name="s")

    @functools.partial(
        pl.kernel,
        mesh=mesh,
        out_type=jax.ShapeDtypeStruct((n_b, n_tok, embed_dim), jnp.float32),
        scratch_types=[
            pltpu.VMEM((chunks, rpg), jnp.int32),
            pltpu.VMEM((NBUF, rpg, embed_dim), jnp.float32),
            pltpu.VMEM_SHARED((embed_num, embed_dim), jnp.float32),
            pltpu.SemaphoreType.DMA((NBUF,)),
            pltpu.SemaphoreType.DMA((NBUF,)),
        ],
        compiler_params=pltpu.CompilerParams(use_tc_tiling_on_sc=False),
    )
    def k(idx_hbm, table_hbm, out_hbm, idx_v, rows_v, table_sh, gsems, osems):
        sid = lax.axis_index("s")
        wid = sid * nc + lax.axis_index("c")
        tslice = embed_num // ns
        pltpu.sync_copy(
            table_hbm.at[pl.ds(sid * tslice, tslice), pl.ds(0, embed_dim)],
            table_sh.at[pl.ds(sid * tslice, tslice)],
        )
        table_2d = table_sh
        pltpu.sync_copy(idx_hbm.at[pl.ds(wid * chunks, chunks)], idx_v)
        plsc.subcore_barrier()
        chunk0 = wid * chunks

        def out_slice(j):
            gc = chunk0 + j
            return out_hbm.at[lax.div(gc, cpb), pl.ds(lax.rem(gc, cpb) * rpg, rpg)]

        for b in range(PREFETCH):
            pltpu.async_copy(table_2d.at[idx_v.at[b]], rows_v.at[b], gsems.at[b])

        def step(j, carry):
            b = lax.rem(j, NBUF)
            b2 = lax.rem(j + PREFETCH, NBUF)

            pltpu.make_async_copy(
                table_2d.at[idx_v.at[j]], rows_v.at[b], gsems.at[b]
            ).wait()

            @pl.when(j + PREFETCH >= NBUF)
            def _():
                pltpu.make_async_copy(
                    rows_v.at[b2], out_slice(j + PREFETCH - NBUF), osems.at[b2]
                ).wait()

            @pl.when(j + PREFETCH < chunks)
            def _():
                pltpu.async_copy(
                    table_2d.at[idx_v.at[j + PREFETCH]], rows_v.at[b2], gsems.at[b2]
                )

            pltpu.async_copy(rows_v.at[b], out_slice(j), osems.at[b])
            return carry

        lax.fori_loop(0, chunks, step, 0)

        for b in range(PREFETCH):
            j = chunks - PREFETCH + b
            pltpu.make_async_copy(
                rows_v.at[j % NBUF], out_slice(j), osems.at[j % NBUF]
            ).wait()

    return k


def kernel(indices, embeddings):
    n_b, n_tok = indices.shape
    embed_num, embed_dim = embeddings.shape
    k = _make_sc_gather(n_b, n_tok, embed_num, embed_dim)
    idx2 = indices.reshape(n_b * n_tok // 128, 128)
    table_padded = jnp.pad(embeddings, ((0, 0), (0, PAD_DIM - embed_dim)))
    return k(idx2, table_padded)


# speedup vs baseline: 1.0003x; 1.0003x over previous
import functools

import jax
import jax.numpy as jnp
from jax import lax
from jax.experimental import pallas as pl
from jax.experimental.pallas import tpu as pltpu
from jax.experimental.pallas import tpu_sc as plsc

ROWS_PER_GATHER = 128
PAD_DIM = 128
NBUF = 8
PREFETCH = 4


@functools.lru_cache(maxsize=None)
def _make_sc_gather(n_b: int, n_tok: int, embed_num: int, embed_dim: int):
    info = plsc.get_sparse_core_info()
    nc, ns = info.num_cores, info.num_subcores
    nw = nc * ns
    n_rows = n_b * n_tok
    rows_per_w = n_rows // nw
    chunks = rows_per_w // ROWS_PER_GATHER
    rpg = ROWS_PER_GATHER
    cpb = n_tok // rpg

    mesh = plsc.VectorSubcoreMesh(core_axis_name="c", subcore_axis_name="s")

    @functools.partial(
        pl.kernel,
        mesh=mesh,
        out_type=jax.ShapeDtypeStruct((n_b, n_tok, embed_dim), jnp.float32),
        scratch_types=[
            pltpu.VMEM((chunks, rpg), jnp.int32),
            pltpu.VMEM((NBUF, rpg, embed_dim), jnp.float32),
            pltpu.VMEM_SHARED((embed_num, embed_dim), jnp.float32),
            pltpu.SemaphoreType.DMA((NBUF,)),
            pltpu.SemaphoreType.DMA((NBUF,)),
        ],
        compiler_params=pltpu.CompilerParams(use_tc_tiling_on_sc=False),
    )
    def k(idx_hbm, table_hbm, out_hbm, idx_v, rows_v, table_sh, gsems, osems):
        sid = lax.axis_index("s")
        wid = sid * nc + lax.axis_index("c")
        tslice = embed_num // ns
        pltpu.sync_copy(
            table_hbm.at[pl.ds(sid * tslice, tslice), pl.ds(0, embed_dim)],
            table_sh.at[pl.ds(sid * tslice, tslice)],
        )
        table_2d = table_sh
        pltpu.sync_copy(idx_hbm.at[pl.ds(wid * chunks, chunks)], idx_v)
        plsc.subcore_barrier()
        chunk0 = wid * chunks

        def out_slice(j):
            gc = chunk0 + j
            return out_hbm.at[lax.div(gc, cpb), pl.ds(lax.rem(gc, cpb) * rpg, rpg)]

        for b in range(PREFETCH):
            pltpu.async_copy(table_2d.at[idx_v.at[b]], rows_v.at[b], gsems.at[b])

        def step(j, carry):
            b = lax.rem(j, NBUF)
            b2 = lax.rem(j + PREFETCH, NBUF)

            pltpu.make_async_copy(
                table_2d.at[idx_v.at[j]], rows_v.at[b], gsems.at[b]
            ).wait()

            @pl.when(j + PREFETCH >= NBUF)
            def _():
                pltpu.make_async_copy(
                    rows_v.at[b2], out_slice(j + PREFETCH - NBUF), osems.at[b2]
                ).wait()

            @pl.when(j + PREFETCH < chunks)
            def _():
                pltpu.async_copy(
                    table_2d.at[idx_v.at[j + PREFETCH]], rows_v.at[b2], gsems.at[b2]
                )

            pltpu.async_copy(rows_v.at[b], out_slice(j), osems.at[b])
            return carry

        lax.fori_loop(0, chunks, step, 0)

        for b in range(PREFETCH):
            j = chunks - PREFETCH + b
            pltpu.make_async_copy(
                rows_v.at[j % NBUF], out_slice(j), osems.at[j % NBUF]
            ).wait()

    return k


def kernel(indices, embeddings):
    n_b, n_tok = indices.shape
    embed_num, embed_dim = embeddings.shape
    k = _make_sc_gather(n_b, n_tok, embed_num, embed_dim)
    idx2 = jnp.maximum(indices.reshape(n_b * n_tok // 128, 128), 0)
    table_padded = jnp.maximum(
        jnp.pad(embeddings, ((0, 0), (0, PAD_DIM - embed_dim))),
        jnp.float32(-3.0e38),
    )
    return k(idx2, table_padded)
